# baseline (speedup 1.0000x reference)
import functools
import math

import jax
import jax.numpy as jnp
from jax.experimental import pallas as pl

MEMORY_SIZE = 100000
KEY_DIM = 16
TOP_K = 256
INV_TEMP = 40
MARGIN = 0.1
SOFTMAX_TEMP = max(1.0, math.log(0.2 * TOP_K) / INV_TEMP)
BATCH = 1024

KB = 2048
NPAD = ((MEMORY_SIZE + KB - 1) // KB) * KB


def _query_body(x_ref, w_ref, b_ref, o_ref):
    q = jax.lax.dot_general(
        x_ref[...].astype(jnp.bfloat16), w_ref[...].astype(jnp.bfloat16),
        (((1,), (1,)), ((), ())),
        preferred_element_type=jnp.float32,
    ) + b_ref[...]
    n = jnp.sqrt(jnp.sum(q * q, axis=1, keepdims=True))
    o_ref[...] = q / jnp.maximum(n, 1e-12)


def _scores_body(q_ref, k_ref, o_ref):
    j = pl.program_id(0)
    s = jax.lax.dot_general(
        q_ref[...].astype(jnp.bfloat16), k_ref[...].astype(jnp.bfloat16),
        (((1,), (1,)), ((), ())),
        preferred_element_type=jnp.float32,
    )
    col = jax.lax.broadcasted_iota(jnp.int32, s.shape, 1) + j * KB
    o_ref[...] = jnp.where(col < MEMORY_SIZE, s, -1e30)


@jax.jit
def _fwd(x, W, b, keys_pad):
    query = pl.pallas_call(
        _query_body,
        out_shape=jax.ShapeDtypeStruct((BATCH, KEY_DIM), jnp.float32),
    )(x, W, b.reshape(1, KEY_DIM))
    scores = pl.pallas_call(
        _scores_body,
        grid=(NPAD // KB,),
        in_specs=[
            pl.BlockSpec((BATCH, KEY_DIM), lambda j: (0, 0)),
            pl.BlockSpec((KB, KEY_DIM), lambda j: (j, 0)),
        ],
        out_specs=pl.BlockSpec((BATCH, KB), lambda j: (0, j)),
        out_shape=jax.ShapeDtypeStruct((BATCH, NPAD), jnp.float32),
    )(query, keys_pad)
    return query, scores


def kernel(x, y, W, b, keys, values):
    keys_pad = jnp.pad(keys, ((0, NPAD - MEMORY_SIZE), (0, 0)))
    query, scores = _fwd(x, W, b, keys_pad)
    cosine_similarity, topk_indices = jax.lax.top_k(scores, TOP_K)
    softmax_score = jax.nn.softmax(SOFTMAX_TEMP * cosine_similarity, axis=1)
    y_hat_indices = topk_indices[:, 0]
    y_hat = jnp.take(values, y_hat_indices, axis=0)
    topk_values = jnp.take(values, topk_indices, axis=0)
    correct_mask = jnp.squeeze(
        jnp.equal(topk_values, y[:, None, :]), axis=-1).astype(jnp.float32)
    pos_score, _ = jax.lax.top_k(cosine_similarity * correct_mask, 1)
    neg_score, _ = jax.lax.top_k(cosine_similarity * (1.0 - correct_mask), 1)
    mask = 1.0 - jnp.equal(jnp.sum(correct_mask, axis=1), 0.0).astype(jnp.float32)
    pos_score = pos_score * mask[:, None]
    dist_hinge = jnp.maximum(neg_score - pos_score + MARGIN, 0.0)
    loss = jnp.mean(dist_hinge)
    return (y_hat, softmax_score, loss, query, y_hat_indices)


# speedup vs baseline: 10.4339x; 10.4339x over previous
import functools
import math

import jax
import jax.numpy as jnp
from jax import lax
from jax.experimental import pallas as pl
from jax.experimental.pallas import tpu as pltpu
from jax.experimental.pallas import tpu_sc as plsc

MEMORY_SIZE = 100000
KEY_DIM = 16
TOP_K = 256
INV_TEMP = 40
MARGIN = 0.1
SOFTMAX_TEMP = max(1.0, math.log(0.2 * TOP_K) / INV_TEMP)
BATCH = 1024

KB = 2048
NPAD = 49 * KB
NBLK = NPAD // 128
CAP = 768
NTILES = 32
ROWS_PER_TILE = BATCH // NTILES


def _query_body(x_ref, w_ref, b_ref, o_ref):
    q = jax.lax.dot_general(
        x_ref[...].astype(jnp.bfloat16), w_ref[...].astype(jnp.bfloat16),
        (((1,), (1,)), ((), ())),
        preferred_element_type=jnp.float32,
    ) + b_ref[...]
    n = jnp.sqrt(jnp.sum(q * q, axis=1, keepdims=True))
    o_ref[...] = q / jnp.maximum(n, 1e-12)


def _scores_body(q_ref, k_ref, o_ref, bm_ref):
    j = pl.program_id(0)
    s = jax.lax.dot_general(
        q_ref[...].astype(jnp.bfloat16), k_ref[...].astype(jnp.bfloat16),
        (((1,), (1,)), ((), ())),
        preferred_element_type=jnp.float32,
    )
    col = jax.lax.broadcasted_iota(jnp.int32, s.shape, 1) + j * KB
    s = jnp.where(col < MEMORY_SIZE, s, -1e30)
    o_ref[...] = s
    bm_ref[0] = jnp.max(s.reshape(BATCH, KB // 128, 128), axis=2)


def _thresh_body(bm_ref, thr_ref):
    bm = bm_ref[...]
    lo0 = jnp.full((1, BATCH, 1), -1.01, jnp.float32)
    hi0 = jnp.full((1, BATCH, 1), 1.01, jnp.float32)

    def it(_, c):
        lo, hi = c
        mid = (lo + hi) * 0.5
        m = (bm >= mid).astype(jnp.float32)
        cnt = jnp.sum(jnp.sum(m, axis=2, keepdims=True), axis=0, keepdims=True)
        ok = cnt >= float(TOP_K)
        return jnp.where(ok, mid, lo), jnp.where(ok, hi, mid)

    lo, _ = lax.fori_loop(0, 22, it, (lo0, hi0))
    thr_ref[...] = jnp.broadcast_to(lo.reshape(BATCH, 1), (BATCH, 16))


def _select_body(scores_hbm, thr_hbm, vals_hbm, idx_hbm, row_v, thr_v, cv_v, ci_v):
    wid = lax.axis_index("s") * 2 + lax.axis_index("c")
    iota16 = lax.iota(jnp.int32, 16)

    def row_body(i, carry):
        r = wid * ROWS_PER_TILE + i
        pltpu.sync_copy(scores_hbm.at[r], row_v)
        pltpu.sync_copy(thr_hbm.at[r], thr_v)
        tvec = thr_v[...]

        def fill(j, c):
            ci_v[pl.ds(j * 16, 16)] = jnp.full((16,), NPAD - 1, jnp.int32)
            return c
        lax.fori_loop(0, CAP // 16, fill, 0)

        def scan4(g, off):
            base = g * 64
            v0 = row_v[pl.ds(base, 16)]
            v1 = row_v[pl.ds(base + 16, 16)]
            v2 = row_v[pl.ds(base + 32, 16)]
            v3 = row_v[pl.ds(base + 48, 16)]
            mx = jnp.maximum(jnp.maximum(v0, v1), jnp.maximum(v2, v3))
            any4 = jnp.any(mx >= tvec)

            def slow(off_in):
                def one(off, v, k):
                    m = v >= tvec
                    idxv = iota16 + (base + k)
                    offc = jnp.minimum(off, CAP - 16)
                    plsc.store_compressed(ci_v.at[pl.ds(offc, 16)], idxv, mask=m)
                    return off + jnp.sum(m.astype(jnp.int32))
                off = one(off_in, v0, 0)
                off = one(off, v1, 16)
                off = one(off, v2, 32)
                off = one(off, v3, 48)
                return off

            return lax.cond(any4, slow, lambda o: o, off)

        lax.fori_loop(0, NPAD // 64, scan4, jnp.int32(0))

        def gath(j, c):
            ii = ci_v[pl.ds(j * 16, 16)]
            cv_v[pl.ds(j * 16, 16)] = plsc.load_gather(row_v, [ii])
            return c
        lax.fori_loop(0, CAP // 16, gath, 0)

        pltpu.sync_copy(cv_v, vals_hbm.at[r])
        pltpu.sync_copy(ci_v, idx_hbm.at[r])
        return carry

    lax.fori_loop(0, ROWS_PER_TILE, row_body, 0)


@jax.jit
def _pipeline(x, y, W, b, keys_pad, values):
    query = pl.pallas_call(
        _query_body,
        out_shape=jax.ShapeDtypeStruct((BATCH, KEY_DIM), jnp.float32),
    )(x, W, b.reshape(1, KEY_DIM))

    scores, bm = pl.pallas_call(
        _scores_body,
        grid=(NPAD // KB,),
        in_specs=[
            pl.BlockSpec((BATCH, KEY_DIM), lambda j: (0, 0)),
            pl.BlockSpec((KB, KEY_DIM), lambda j: (j, 0)),
        ],
        out_specs=[
            pl.BlockSpec((BATCH, KB), lambda j: (0, j)),
            pl.BlockSpec((1, BATCH, KB // 128), lambda j: (j, 0, 0)),
        ],
        out_shape=[
            jax.ShapeDtypeStruct((BATCH, NPAD), jnp.float32),
            jax.ShapeDtypeStruct((NPAD // KB, BATCH, KB // 128), jnp.float32),
        ],
    )(query, keys_pad)

    thr = pl.pallas_call(
        _thresh_body,
        out_shape=jax.ShapeDtypeStruct((BATCH, 16), jnp.float32),
    )(bm)

    mesh = plsc.VectorSubcoreMesh(
        core_axis_name="c", subcore_axis_name="s", num_cores=2, num_subcores=16)
    cand_vals, cand_idx = pl.kernel(
        _select_body,
        out_type=[
            jax.ShapeDtypeStruct((BATCH, CAP), jnp.float32),
            jax.ShapeDtypeStruct((BATCH, CAP), jnp.int32),
        ],
        mesh=mesh,
        compiler_params=pltpu.CompilerParams(needs_layout_passes=False),
        scratch_types=[
            pltpu.VMEM((NPAD,), jnp.float32),
            pltpu.VMEM((16,), jnp.float32),
            pltpu.VMEM((CAP,), jnp.float32),
            pltpu.VMEM((CAP,), jnp.int32),
        ],
    )(scores, thr)

    cosine_similarity, pos = jax.lax.top_k(cand_vals, TOP_K)
    topk_indices = jnp.take_along_axis(cand_idx, pos, axis=1)

    softmax_score = jax.nn.softmax(SOFTMAX_TEMP * cosine_similarity, axis=1)
    y_hat_indices = topk_indices[:, 0]
    y_hat = jnp.take(values, y_hat_indices, axis=0)
    topk_values = jnp.take(values, topk_indices, axis=0)
    correct_mask = jnp.squeeze(
        jnp.equal(topk_values, y[:, None, :]), axis=-1).astype(jnp.float32)
    pos_score, _ = jax.lax.top_k(cosine_similarity * correct_mask, 1)
    neg_score, _ = jax.lax.top_k(cosine_similarity * (1.0 - correct_mask), 1)
    mask = 1.0 - jnp.equal(jnp.sum(correct_mask, axis=1), 0.0).astype(jnp.float32)
    pos_score = pos_score * mask[:, None]
    dist_hinge = jnp.maximum(neg_score - pos_score + MARGIN, 0.0)
    loss = jnp.mean(dist_hinge)
    return (y_hat, softmax_score, loss, query, y_hat_indices)


def kernel(x, y, W, b, keys, values):
    keys_pad = jnp.pad(keys, ((0, NPAD - MEMORY_SIZE), (0, 0)))
    return _pipeline(x, y, W, b, keys_pad, values)


# speedup vs baseline: 12.3510x; 1.1837x over previous
import functools
import math

import jax
import jax.numpy as jnp
from jax import lax
from jax.experimental import pallas as pl
from jax.experimental.pallas import tpu as pltpu
from jax.experimental.pallas import tpu_sc as plsc

MEMORY_SIZE = 100000
KEY_DIM = 16
TOP_K = 256
INV_TEMP = 40
MARGIN = 0.1
SOFTMAX_TEMP = max(1.0, math.log(0.2 * TOP_K) / INV_TEMP)
BATCH = 1024

KB = 2048
NPAD = 49 * KB
NBLK = NPAD // 128
CAP = 1024
NTILES = 32
ROWS_PER_TILE = BATCH // NTILES


def _query_body(x_ref, w_ref, b_ref, o_ref):
    q = jax.lax.dot_general(
        x_ref[...].astype(jnp.bfloat16), w_ref[...].astype(jnp.bfloat16),
        (((1,), (1,)), ((), ())),
        preferred_element_type=jnp.float32,
    ) + b_ref[...]
    n = jnp.sqrt(jnp.sum(q * q, axis=1, keepdims=True))
    o_ref[...] = q / jnp.maximum(n, 1e-12)


def _scores_body(q_ref, k_ref, o_ref, bm_ref):
    j = pl.program_id(0)
    s = jax.lax.dot_general(
        q_ref[...].astype(jnp.bfloat16), k_ref[...].astype(jnp.bfloat16),
        (((1,), (1,)), ((), ())),
        preferred_element_type=jnp.float32,
    )
    col = jax.lax.broadcasted_iota(jnp.int32, s.shape, 1) + j * KB
    s = jnp.where(col < MEMORY_SIZE, s, -1e30)
    o_ref[...] = s
    bm_ref[0] = jnp.max(s.reshape(BATCH, KB // 128, 128), axis=2)


def _thresh_body(bm_ref, thr_ref):
    bm = bm_ref[...]
    lo0 = jnp.full((1, BATCH, 1), -1.01, jnp.float32)
    hi0 = jnp.full((1, BATCH, 1), 1.01, jnp.float32)

    def it(_, c):
        lo, hi = c
        mid = (lo + hi) * 0.5
        m = (bm >= mid).astype(jnp.float32)
        cnt = jnp.sum(jnp.sum(m, axis=2, keepdims=True), axis=0, keepdims=True)
        ok = cnt >= float(TOP_K)
        return jnp.where(ok, mid, lo), jnp.where(ok, hi, mid)

    lo, _ = lax.fori_loop(0, 22, it, (lo0, hi0))
    thr_ref[...] = jnp.broadcast_to(lo.reshape(BATCH, 1), (BATCH, 16))


def _select_body(scores_hbm, thr_hbm, vals_hbm, idx_hbm, row_v, thr_v, cv_v, ci_v):
    wid = lax.axis_index("s") * 2 + lax.axis_index("c")
    iota16 = lax.iota(jnp.int32, 16)

    def row_body(i, carry):
        r = wid * ROWS_PER_TILE + i
        pltpu.sync_copy(scores_hbm.at[r], row_v)
        pltpu.sync_copy(thr_hbm.at[r], thr_v)
        tvec = thr_v[...]

        def fill(j, c):
            ci_v[pl.ds(j * 16, 16)] = jnp.full((16,), NPAD - 1, jnp.int32)
            return c
        lax.fori_loop(0, CAP // 16, fill, 0)

        def scan4(g, off):
            base = g * 64
            v0 = row_v[pl.ds(base, 16)]
            v1 = row_v[pl.ds(base + 16, 16)]
            v2 = row_v[pl.ds(base + 32, 16)]
            v3 = row_v[pl.ds(base + 48, 16)]
            mx = jnp.maximum(jnp.maximum(v0, v1), jnp.maximum(v2, v3))
            any4 = jnp.any(mx >= tvec)

            def slow(off_in):
                def one(off, v, k):
                    m = v >= tvec
                    idxv = iota16 + (base + k)
                    offc = jnp.minimum(off, CAP - 16)
                    plsc.store_compressed(ci_v.at[pl.ds(offc, 16)], idxv, mask=m)
                    return off + jnp.sum(m.astype(jnp.int32))
                off = one(off_in, v0, 0)
                off = one(off, v1, 16)
                off = one(off, v2, 32)
                off = one(off, v3, 48)
                return off

            return lax.cond(any4, slow, lambda o: o, off)

        lax.fori_loop(0, NPAD // 64, scan4, jnp.int32(0))

        def gath(j, c):
            ii = ci_v[pl.ds(j * 16, 16)]
            cv_v[pl.ds(j * 16, 16)] = plsc.load_gather(row_v, [ii])
            return c
        lax.fori_loop(0, CAP // 16, gath, 0)

        pltpu.sync_copy(cv_v, vals_hbm.at[r])
        pltpu.sync_copy(ci_v, idx_hbm.at[r])
        return carry

    lax.fori_loop(0, ROWS_PER_TILE, row_body, 0)


def _sort_body(cv_ref, ci_ref, cos_ref, idx_ref):
    v = cv_ref[...]
    ix = ci_ref[...]
    n = CAP
    lane = jax.lax.broadcasted_iota(jnp.int32, (1, n), 1)
    k = 2
    while k <= n:
        j = k // 2
        while j >= 1:
            lower = (lane & j) == 0
            first = ((lane & k) == 0) == lower
            pv = jnp.where(lower,
                           jnp.concatenate([v[:, j:], v[:, :j]], axis=1),
                           jnp.concatenate([v[:, n - j:], v[:, :n - j]], axis=1))
            pi = jnp.where(lower,
                           jnp.concatenate([ix[:, j:], ix[:, :j]], axis=1),
                           jnp.concatenate([ix[:, n - j:], ix[:, :n - j]], axis=1))
            c = (v > pv) | ((v == pv) & (ix < pi))
            v = jnp.where(c == first, v, pv)
            ix = jnp.where(c == first, ix, pi)
            j //= 2
        k *= 2
    cos_ref[...] = v[:, :TOP_K]
    idx_ref[...] = ix[:, :TOP_K]


@jax.jit
def _pipeline(x, y, W, b, keys_pad, values):
    query = pl.pallas_call(
        _query_body,
        out_shape=jax.ShapeDtypeStruct((BATCH, KEY_DIM), jnp.float32),
    )(x, W, b.reshape(1, KEY_DIM))

    scores, bm = pl.pallas_call(
        _scores_body,
        grid=(NPAD // KB,),
        in_specs=[
            pl.BlockSpec((BATCH, KEY_DIM), lambda j: (0, 0)),
            pl.BlockSpec((KB, KEY_DIM), lambda j: (j, 0)),
        ],
        out_specs=[
            pl.BlockSpec((BATCH, KB), lambda j: (0, j)),
            pl.BlockSpec((1, BATCH, KB // 128), lambda j: (j, 0, 0)),
        ],
        out_shape=[
            jax.ShapeDtypeStruct((BATCH, NPAD), jnp.float32),
            jax.ShapeDtypeStruct((NPAD // KB, BATCH, KB // 128), jnp.float32),
        ],
    )(query, keys_pad)

    thr = pl.pallas_call(
        _thresh_body,
        out_shape=jax.ShapeDtypeStruct((BATCH, 16), jnp.float32),
    )(bm)

    mesh = plsc.VectorSubcoreMesh(
        core_axis_name="c", subcore_axis_name="s", num_cores=2, num_subcores=16)
    cand_vals, cand_idx = pl.kernel(
        _select_body,
        out_type=[
            jax.ShapeDtypeStruct((BATCH, CAP), jnp.float32),
            jax.ShapeDtypeStruct((BATCH, CAP), jnp.int32),
        ],
        mesh=mesh,
        compiler_params=pltpu.CompilerParams(needs_layout_passes=False),
        scratch_types=[
            pltpu.VMEM((NPAD,), jnp.float32),
            pltpu.VMEM((16,), jnp.float32),
            pltpu.VMEM((CAP,), jnp.float32),
            pltpu.VMEM((CAP,), jnp.int32),
        ],
    )(scores, thr)

    cosine_similarity, topk_indices = pl.pallas_call(
        _sort_body,
        out_shape=[
            jax.ShapeDtypeStruct((BATCH, TOP_K), jnp.float32),
            jax.ShapeDtypeStruct((BATCH, TOP_K), jnp.int32),
        ],
    )(cand_vals, cand_idx)

    softmax_score = jax.nn.softmax(SOFTMAX_TEMP * cosine_similarity, axis=1)
    y_hat_indices = topk_indices[:, 0]
    y_hat = jnp.take(values, y_hat_indices, axis=0)
    topk_values = jnp.take(values, topk_indices, axis=0)
    correct_mask = jnp.squeeze(
        jnp.equal(topk_values, y[:, None, :]), axis=-1).astype(jnp.float32)
    pos_score, _ = jax.lax.top_k(cosine_similarity * correct_mask, 1)
    neg_score, _ = jax.lax.top_k(cosine_similarity * (1.0 - correct_mask), 1)
    mask = 1.0 - jnp.equal(jnp.sum(correct_mask, axis=1), 0.0).astype(jnp.float32)
    pos_score = pos_score * mask[:, None]
    dist_hinge = jnp.maximum(neg_score - pos_score + MARGIN, 0.0)
    loss = jnp.mean(dist_hinge)
    return (y_hat, softmax_score, loss, query, y_hat_indices)


def kernel(x, y, W, b, keys, values):
    keys_pad = jnp.pad(keys, ((0, NPAD - MEMORY_SIZE), (0, 0)))
    return _pipeline(x, y, W, b, keys_pad, values)


# speedup vs baseline: 15.5681x; 1.2605x over previous
import functools
import math

import jax
import jax.numpy as jnp
from jax import lax
from jax.experimental import pallas as pl
from jax.experimental.pallas import tpu as pltpu
from jax.experimental.pallas import tpu_sc as plsc

MEMORY_SIZE = 100000
KEY_DIM = 16
TOP_K = 256
INV_TEMP = 40
MARGIN = 0.1
SOFTMAX_TEMP = max(1.0, math.log(0.2 * TOP_K) / INV_TEMP)
BATCH = 1024

KB = 2048
NPAD = 49 * KB
NBLK = NPAD // 128
CAP = 1024
NTILES = 32
ROWS_PER_TILE = BATCH // NTILES


def _query_body(x_ref, w_ref, b_ref, o_ref):
    q = jax.lax.dot_general(
        x_ref[...].astype(jnp.bfloat16), w_ref[...].astype(jnp.bfloat16),
        (((1,), (1,)), ((), ())),
        preferred_element_type=jnp.float32,
    ) + b_ref[...]
    n = jnp.sqrt(jnp.sum(q * q, axis=1, keepdims=True))
    o_ref[...] = q / jnp.maximum(n, 1e-12)


def _scores_body(q_ref, k_ref, o_ref, bm_ref):
    j = pl.program_id(0)
    s = jax.lax.dot_general(
        q_ref[...].astype(jnp.bfloat16), k_ref[...].astype(jnp.bfloat16),
        (((1,), (1,)), ((), ())),
        preferred_element_type=jnp.float32,
    )
    col = jax.lax.broadcasted_iota(jnp.int32, s.shape, 1) + j * KB
    s = jnp.where(col < MEMORY_SIZE, s, -1e30)
    o_ref[...] = s
    bm_ref[0] = jnp.max(s.reshape(BATCH, KB // 128, 128), axis=2)


def _thresh_body(bm_ref, thr_ref, bmr_ref):
    bm = bm_ref[...]
    lo0 = jnp.full((1, BATCH, 1), -1.01, jnp.float32)
    hi0 = jnp.full((1, BATCH, 1), 1.01, jnp.float32)

    def it(_, c):
        lo, hi = c
        mid = (lo + hi) * 0.5
        m = (bm >= mid).astype(jnp.float32)
        cnt = jnp.sum(jnp.sum(m, axis=2, keepdims=True), axis=0, keepdims=True)
        ok = cnt >= float(TOP_K)
        return jnp.where(ok, mid, lo), jnp.where(ok, hi, mid)

    lo, _ = lax.fori_loop(0, 22, it, (lo0, hi0))
    thr_ref[...] = jnp.broadcast_to(lo.reshape(BATCH, 1), (BATCH, 16))
    for j in range(NPAD // KB):
        bmr_ref[:, j * 16:(j + 1) * 16] = bm[j]


BCAP = 512


def _select_body(scores2_hbm, bmr_hbm, thr_hbm, vals_hbm, idx_hbm,
                 bm_v, thr_v, blk_v, gath_v, cv_v, ci_v, sem):
    wid = lax.axis_index("s") * 2 + lax.axis_index("c")
    iota16 = lax.iota(jnp.int32, 16)

    def row_body(i, carry):
        r = wid * ROWS_PER_TILE + i
        rbase = r * NBLK
        pltpu.sync_copy(bmr_hbm.at[r], bm_v)
        pltpu.sync_copy(thr_hbm.at[r], thr_v)
        tvec = thr_v[...]

        def fillb(j, c):
            blk_v[pl.ds(j * 16, 16)] = (iota16 + j * 16) * NBLK + (NBLK - 1)
            return c
        lax.fori_loop(0, BCAP // 16, fillb, 0)

        def fillc(j, c):
            cv_v[pl.ds(j * 16, 16)] = jnp.full((16,), -1e30, jnp.float32)
            ci_v[pl.ds(j * 16, 16)] = jnp.full((16,), NPAD - 1, jnp.int32)
            return c
        lax.fori_loop(0, CAP // 16, fillc, 0)

        def blkscan(s, boff):
            m = bm_v[pl.ds(s * 16, 16)] >= tvec
            idxv = iota16 + (s * 16) + rbase
            offc = jnp.minimum(boff, BCAP - 16)
            plsc.store_compressed(blk_v.at[pl.ds(offc, 16)], idxv, mask=m)
            return boff + jnp.sum(m.astype(jnp.int32))
        bcnt = lax.fori_loop(0, NBLK // 16, blkscan, jnp.int32(0))
        bcnt = jnp.minimum(bcnt, BCAP)

        pltpu.async_copy(scores2_hbm.at[blk_v], gath_v, sem).wait()

        def extract(s, off):
            bvec = plsc.load_gather(blk_v, [jnp.zeros((16,), jnp.int32) + s])
            base = (bvec - rbase) * 128
            for t in range(8):
                v = gath_v[s, pl.ds(t * 16, 16)]
                m = v >= tvec
                idxv = base + (t * 16) + iota16
                offc = jnp.minimum(off, CAP - 16)
                plsc.store_compressed(cv_v.at[pl.ds(offc, 16)], v, mask=m)
                plsc.store_compressed(ci_v.at[pl.ds(offc, 16)], idxv, mask=m)
                off = off + jnp.sum(m.astype(jnp.int32))
            return off
        lax.fori_loop(0, bcnt, extract, jnp.int32(0))

        pltpu.sync_copy(cv_v, vals_hbm.at[r])
        pltpu.sync_copy(ci_v, idx_hbm.at[r])
        return carry

    lax.fori_loop(0, ROWS_PER_TILE, row_body, 0)


def _sort_body(cv_ref, ci_ref, cos_ref, idx_ref):
    v = cv_ref[...]
    ix = ci_ref[...]
    n = CAP
    lane = jax.lax.broadcasted_iota(jnp.int32, (1, n), 1)
    k = 2
    while k <= n:
        j = k // 2
        while j >= 1:
            lower = (lane & j) == 0
            first = ((lane & k) == 0) == lower
            pv = jnp.where(lower,
                           jnp.concatenate([v[:, j:], v[:, :j]], axis=1),
                           jnp.concatenate([v[:, n - j:], v[:, :n - j]], axis=1))
            pi = jnp.where(lower,
                           jnp.concatenate([ix[:, j:], ix[:, :j]], axis=1),
                           jnp.concatenate([ix[:, n - j:], ix[:, :n - j]], axis=1))
            c = (v > pv) | ((v == pv) & (ix < pi))
            v = jnp.where(c == first, v, pv)
            ix = jnp.where(c == first, ix, pi)
            j //= 2
        k *= 2
    cos_ref[...] = v[:, :TOP_K]
    idx_ref[...] = ix[:, :TOP_K]


@jax.jit
def _pipeline(x, y, W, b, keys_pad, values):
    query = pl.pallas_call(
        _query_body,
        out_shape=jax.ShapeDtypeStruct((BATCH, KEY_DIM), jnp.float32),
    )(x, W, b.reshape(1, KEY_DIM))

    scores, bm = pl.pallas_call(
        _scores_body,
        grid=(NPAD // KB,),
        in_specs=[
            pl.BlockSpec((BATCH, KEY_DIM), lambda j: (0, 0)),
            pl.BlockSpec((KB, KEY_DIM), lambda j: (j, 0)),
        ],
        out_specs=[
            pl.BlockSpec((BATCH, KB), lambda j: (0, j)),
            pl.BlockSpec((1, BATCH, KB // 128), lambda j: (j, 0, 0)),
        ],
        out_shape=[
            jax.ShapeDtypeStruct((BATCH, NPAD), jnp.float32),
            jax.ShapeDtypeStruct((NPAD // KB, BATCH, KB // 128), jnp.float32),
        ],
    )(query, keys_pad)

    thr, bmr = pl.pallas_call(
        _thresh_body,
        out_shape=[
            jax.ShapeDtypeStruct((BATCH, 16), jnp.float32),
            jax.ShapeDtypeStruct((BATCH, NBLK), jnp.float32),
        ],
    )(bm)

    scores2 = scores.reshape(BATCH * NBLK, 128)
    mesh = plsc.VectorSubcoreMesh(
        core_axis_name="c", subcore_axis_name="s", num_cores=2, num_subcores=16)
    cand_vals, cand_idx = pl.kernel(
        _select_body,
        out_type=[
            jax.ShapeDtypeStruct((BATCH, CAP), jnp.float32),
            jax.ShapeDtypeStruct((BATCH, CAP), jnp.int32),
        ],
        mesh=mesh,
        compiler_params=pltpu.CompilerParams(needs_layout_passes=False),
        scratch_types=[
            pltpu.VMEM((NBLK,), jnp.float32),
            pltpu.VMEM((16,), jnp.float32),
            pltpu.VMEM((BCAP,), jnp.int32),
            pltpu.VMEM((BCAP, 128), jnp.float32),
            pltpu.VMEM((CAP,), jnp.float32),
            pltpu.VMEM((CAP,), jnp.int32),
            pltpu.SemaphoreType.DMA,
        ],
    )(scores2, bmr, thr)

    cosine_similarity, topk_indices = pl.pallas_call(
        _sort_body,
        out_shape=[
            jax.ShapeDtypeStruct((BATCH, TOP_K), jnp.float32),
            jax.ShapeDtypeStruct((BATCH, TOP_K), jnp.int32),
        ],
    )(cand_vals, cand_idx)

    softmax_score = jax.nn.softmax(SOFTMAX_TEMP * cosine_similarity, axis=1)
    y_hat_indices = topk_indices[:, 0]
    y_hat = jnp.take(values, y_hat_indices, axis=0)
    topk_values = jnp.take(values, topk_indices, axis=0)
    correct_mask = jnp.squeeze(
        jnp.equal(topk_values, y[:, None, :]), axis=-1).astype(jnp.float32)
    pos_score, _ = jax.lax.top_k(cosine_similarity * correct_mask, 1)
    neg_score, _ = jax.lax.top_k(cosine_similarity * (1.0 - correct_mask), 1)
    mask = 1.0 - jnp.equal(jnp.sum(correct_mask, axis=1), 0.0).astype(jnp.float32)
    pos_score = pos_score * mask[:, None]
    dist_hinge = jnp.maximum(neg_score - pos_score + MARGIN, 0.0)
    loss = jnp.mean(dist_hinge)
    return (y_hat, softmax_score, loss, query, y_hat_indices)


def kernel(x, y, W, b, keys, values):
    keys_pad = jnp.pad(keys, ((0, NPAD - MEMORY_SIZE), (0, 0)))
    return _pipeline(x, y, W, b, keys_pad, values)


# speedup vs baseline: 16.3223x; 1.0484x over previous
import functools
import math

import jax
import jax.numpy as jnp
from jax import lax
from jax.experimental import pallas as pl
from jax.experimental.pallas import tpu as pltpu
from jax.experimental.pallas import tpu_sc as plsc

MEMORY_SIZE = 100000
KEY_DIM = 16
TOP_K = 256
INV_TEMP = 40
MARGIN = 0.1
SOFTMAX_TEMP = max(1.0, math.log(0.2 * TOP_K) / INV_TEMP)
BATCH = 1024

KB = 2048
NPAD = 49 * KB
NBLK = NPAD // 128
CAP = 512
NTILES = 32
ROWS_PER_TILE = BATCH // NTILES


def _query_body(x_ref, w_ref, b_ref, o_ref):
    q = jax.lax.dot_general(
        x_ref[...].astype(jnp.bfloat16), w_ref[...].astype(jnp.bfloat16),
        (((1,), (1,)), ((), ())),
        preferred_element_type=jnp.float32,
    ) + b_ref[...]
    n = jnp.sqrt(jnp.sum(q * q, axis=1, keepdims=True))
    o_ref[...] = q / jnp.maximum(n, 1e-12)


def _scores_body(q_ref, k_ref, o_ref, bm_ref):
    j = pl.program_id(0)
    s = jax.lax.dot_general(
        q_ref[...].astype(jnp.bfloat16), k_ref[...].astype(jnp.bfloat16),
        (((1,), (1,)), ((), ())),
        preferred_element_type=jnp.float32,
    )
    col = jax.lax.broadcasted_iota(jnp.int32, s.shape, 1) + j * KB
    s = jnp.where(col < MEMORY_SIZE, s, -1e30)
    o_ref[...] = s
    bm_ref[0] = jnp.max(s.reshape(BATCH, KB // 128, 128), axis=2)


def _thresh_body(bm_ref, thr_ref, bmr_ref):
    bm = bm_ref[...]
    lo0 = jnp.full((1, BATCH, 1), -1.01, jnp.float32)
    hi0 = jnp.full((1, BATCH, 1), 1.01, jnp.float32)

    def it(_, c):
        lo, hi = c
        mid = (lo + hi) * 0.5
        m = (bm >= mid).astype(jnp.float32)
        cnt = jnp.sum(jnp.sum(m, axis=2, keepdims=True), axis=0, keepdims=True)
        ok = cnt >= float(TOP_K)
        return jnp.where(ok, mid, lo), jnp.where(ok, hi, mid)

    lo, _ = lax.fori_loop(0, 22, it, (lo0, hi0))
    thr_ref[...] = jnp.broadcast_to(lo.reshape(BATCH, 1), (BATCH, 16))
    for j in range(NPAD // KB):
        bmr_ref[:, j * 16:(j + 1) * 16] = bm[j]


BCAP = 512


def _select_body(scores2_hbm, bmr_hbm, thr_hbm, vals_hbm, idx_hbm,
                 bm_v, thr_v, blk_v, gath_v, cv_v, ci_v, sem):
    wid = lax.axis_index("s") * 2 + lax.axis_index("c")
    iota16 = lax.iota(jnp.int32, 16)

    def row_body(i, carry):
        r = wid * ROWS_PER_TILE + i
        rbase = r * NBLK
        pltpu.sync_copy(bmr_hbm.at[r], bm_v)
        pltpu.sync_copy(thr_hbm.at[r], thr_v)
        tvec = thr_v[...]

        def fillb(j, c):
            blk_v[pl.ds(j * 16, 16)] = (iota16 + j * 16) * NBLK + (NBLK - 1)
            return c
        lax.fori_loop(0, BCAP // 16, fillb, 0)

        def fillc(j, c):
            cv_v[pl.ds(j * 16, 16)] = jnp.full((16,), -1e30, jnp.float32)
            ci_v[pl.ds(j * 16, 16)] = jnp.full((16,), NPAD - 1, jnp.int32)
            return c
        lax.fori_loop(0, CAP // 16, fillc, 0)

        def blkscan(s, boff):
            m = bm_v[pl.ds(s * 16, 16)] >= tvec
            idxv = iota16 + (s * 16) + rbase
            offc = jnp.minimum(boff, BCAP - 16)
            plsc.store_compressed(blk_v.at[pl.ds(offc, 16)], idxv, mask=m)
            return boff + jnp.sum(m.astype(jnp.int32))
        bcnt = lax.fori_loop(0, NBLK // 16, blkscan, jnp.int32(0))
        bcnt = jnp.minimum(bcnt, BCAP)

        pltpu.async_copy(scores2_hbm.at[blk_v], gath_v, sem).wait()

        def extract(s, off):
            bvec = plsc.load_gather(blk_v, [jnp.zeros((16,), jnp.int32) + s])
            base = (bvec - rbase) * 128
            for t in range(8):
                v = gath_v[s, pl.ds(t * 16, 16)]
                m = v >= tvec
                idxv = base + (t * 16) + iota16
                offc = jnp.minimum(off, CAP - 16)
                plsc.store_compressed(cv_v.at[pl.ds(offc, 16)], v, mask=m)
                plsc.store_compressed(ci_v.at[pl.ds(offc, 16)], idxv, mask=m)
                off = off + jnp.sum(m.astype(jnp.int32))
            return off
        lax.fori_loop(0, bcnt, extract, jnp.int32(0))

        pltpu.sync_copy(cv_v, vals_hbm.at[r])
        pltpu.sync_copy(ci_v, idx_hbm.at[r])
        return carry

    lax.fori_loop(0, ROWS_PER_TILE, row_body, 0)


def _sort_body(cv_ref, ci_ref, cos_ref, idx_ref):
    v = cv_ref[...]
    ix = ci_ref[...]
    n = CAP
    lane = jax.lax.broadcasted_iota(jnp.int32, (1, n), 1)
    k = 2
    while k <= n:
        j = k // 2
        while j >= 1:
            lower = (lane & j) == 0
            first = ((lane & k) == 0) == lower
            pv = jnp.where(lower,
                           jnp.concatenate([v[:, j:], v[:, :j]], axis=1),
                           jnp.concatenate([v[:, n - j:], v[:, :n - j]], axis=1))
            pi = jnp.where(lower,
                           jnp.concatenate([ix[:, j:], ix[:, :j]], axis=1),
                           jnp.concatenate([ix[:, n - j:], ix[:, :n - j]], axis=1))
            c = (v > pv) | ((v == pv) & (ix < pi))
            v = jnp.where(c == first, v, pv)
            ix = jnp.where(c == first, ix, pi)
            j //= 2
        k *= 2
    cos_ref[...] = v[:, :TOP_K]
    idx_ref[...] = ix[:, :TOP_K]


@jax.jit
def _pipeline(x, y, W, b, keys_pad, values):
    query = pl.pallas_call(
        _query_body,
        out_shape=jax.ShapeDtypeStruct((BATCH, KEY_DIM), jnp.float32),
    )(x, W, b.reshape(1, KEY_DIM))

    scores, bm = pl.pallas_call(
        _scores_body,
        grid=(NPAD // KB,),
        in_specs=[
            pl.BlockSpec((BATCH, KEY_DIM), lambda j: (0, 0)),
            pl.BlockSpec((KB, KEY_DIM), lambda j: (j, 0)),
        ],
        out_specs=[
            pl.BlockSpec((BATCH, KB), lambda j: (0, j)),
            pl.BlockSpec((1, BATCH, KB // 128), lambda j: (j, 0, 0)),
        ],
        out_shape=[
            jax.ShapeDtypeStruct((BATCH, NPAD), jnp.float32),
            jax.ShapeDtypeStruct((NPAD // KB, BATCH, KB // 128), jnp.float32),
        ],
    )(query, keys_pad)

    thr, bmr = pl.pallas_call(
        _thresh_body,
        out_shape=[
            jax.ShapeDtypeStruct((BATCH, 16), jnp.float32),
            jax.ShapeDtypeStruct((BATCH, NBLK), jnp.float32),
        ],
    )(bm)

    scores2 = scores.reshape(BATCH * NBLK, 128)
    mesh = plsc.VectorSubcoreMesh(
        core_axis_name="c", subcore_axis_name="s", num_cores=2, num_subcores=16)
    cand_vals, cand_idx = pl.kernel(
        _select_body,
        out_type=[
            jax.ShapeDtypeStruct((BATCH, CAP), jnp.float32),
            jax.ShapeDtypeStruct((BATCH, CAP), jnp.int32),
        ],
        mesh=mesh,
        compiler_params=pltpu.CompilerParams(needs_layout_passes=False),
        scratch_types=[
            pltpu.VMEM((NBLK,), jnp.float32),
            pltpu.VMEM((16,), jnp.float32),
            pltpu.VMEM((BCAP,), jnp.int32),
            pltpu.VMEM((BCAP, 128), jnp.float32),
            pltpu.VMEM((CAP,), jnp.float32),
            pltpu.VMEM((CAP,), jnp.int32),
            pltpu.SemaphoreType.DMA,
        ],
    )(scores2, bmr, thr)

    cosine_similarity, topk_indices = pl.pallas_call(
        _sort_body,
        out_shape=[
            jax.ShapeDtypeStruct((BATCH, TOP_K), jnp.float32),
            jax.ShapeDtypeStruct((BATCH, TOP_K), jnp.int32),
        ],
    )(cand_vals, cand_idx)

    softmax_score = jax.nn.softmax(SOFTMAX_TEMP * cosine_similarity, axis=1)
    y_hat_indices = topk_indices[:, 0]
    y_hat = jnp.take(values, y_hat_indices, axis=0)
    topk_values = jnp.take(values, topk_indices, axis=0)
    correct_mask = jnp.squeeze(
        jnp.equal(topk_values, y[:, None, :]), axis=-1).astype(jnp.float32)
    pos_score, _ = jax.lax.top_k(cosine_similarity * correct_mask, 1)
    neg_score, _ = jax.lax.top_k(cosine_similarity * (1.0 - correct_mask), 1)
    mask = 1.0 - jnp.equal(jnp.sum(correct_mask, axis=1), 0.0).astype(jnp.float32)
    pos_score = pos_score * mask[:, None]
    dist_hinge = jnp.maximum(neg_score - pos_score + MARGIN, 0.0)
    loss = jnp.mean(dist_hinge)
    return (y_hat, softmax_score, loss, query, y_hat_indices)


def kernel(x, y, W, b, keys, values):
    keys_pad = jnp.pad(keys, ((0, NPAD - MEMORY_SIZE), (0, 0)))
    return _pipeline(x, y, W, b, keys_pad, values)
